# baseline (speedup 1.0000x reference)
import functools

import jax
import jax.numpy as jnp
from jax import lax
from jax.experimental import pallas as pl
from jax.experimental.pallas import tpu as pltpu
from jax.experimental.pallas import tpu_sc as plsc

LAT, LON, NQ = 1801, 3600, 1048576
NC, NS, L = 2, 16, 16
NW = NC * NS
QPW = NQ // NW
CH = 4096
NCH = QPW // CH

def _buf_set():
    return (
        [pltpu.VMEM((CH,), jnp.float32)] * 2
        + [pltpu.VMEM((CH,), jnp.int32)] * 2
        + [pltpu.VMEM((CH,), jnp.float32)] * 2
        + [pltpu.VMEM((CH, 8), jnp.float32)]
    )


def _make_interp():
    mesh = plsc.VectorSubcoreMesh(core_axis_name="c", subcore_axis_name="s")

    @functools.partial(
        pl.kernel,
        out_type=jax.ShapeDtypeStruct((NQ,), jnp.float32),
        mesh=mesh,
        compiler_params=pltpu.CompilerParams(
            needs_layout_passes=False, use_tc_tiling_on_sc=False),
        scratch_types=[
            _buf_set(),
            _buf_set(),
            pltpu.VMEM((CH,), jnp.float32),
            pltpu.SemaphoreType.DMA,
            pltpu.SemaphoreType.DMA,
        ],
    )
    def interp(octos, qlat, qlon, out, bufs_a, bufs_b, res_v, sem_a, sem_b):
        wid = lax.axis_index("s") * NC + lax.axis_index("c")
        base = wid * QPW

        def prep(c, bufs):
            xq_v, yq_v, row_v, cb_v, t_v, u_v, _ = bufs
            off = base + c * CH
            pltpu.sync_copy(qlat.at[pl.ds(off, CH)], xq_v)
            pltpu.sync_copy(qlon.at[pl.ds(off, CH)], yq_v)

            @pl.loop(0, CH // L, unroll=4)
            def _idx(kv):
                s = pl.ds(kv * L, L)
                fx = (xq_v[s] + 90.0) * 10.0
                fy = (yq_v[s] + 180.0) * 10.0
                ix = jnp.minimum(fx.astype(jnp.int32), LAT - 2)
                jy = jnp.minimum(fy.astype(jnp.int32), LON - 1)
                t_v[s] = fx - ix.astype(jnp.float32)
                u_v[s] = fy - jy.astype(jnp.float32)
                f00 = ix * LON + jy
                row_v[s] = f00 >> 1
                cb_v[s] = (f00 & 1) << 2

        def copy(bufs, sem):
            row_v, gq_v = bufs[2], bufs[6]
            return pltpu.make_async_copy(octos.at[row_v], gq_v, sem)

        def drain(c, bufs, sem):
            copy(bufs, sem).wait()
            cb_v, t_v, u_v, gq_v = bufs[3], bufs[4], bufs[5], bufs[6]

            @pl.loop(0, CH // L, unroll=4)
            def _blend(kv):
                s = pl.ds(kv * L, L)
                q = lax.iota(jnp.int32, L) + kv * L
                cb = cb_v[s]
                v00 = plsc.load_gather(gq_v, [q, cb])
                v01 = plsc.load_gather(gq_v, [q, cb + 1])
                v10 = plsc.load_gather(gq_v, [q, cb + 2])
                v11 = plsc.load_gather(gq_v, [q, cb + 3])
                t = t_v[s]
                u = u_v[s]
                res_v[s] = ((1.0 - t) * (1.0 - u) * v00
                            + (1.0 - t) * u * v01
                            + t * (1.0 - u) * v10
                            + t * u * v11)

            pltpu.sync_copy(res_v, out.at[pl.ds(base + c * CH, CH)])

        prep(0, bufs_a)
        copy(bufs_a, sem_a).start()

        @pl.loop(0, NCH // 2)
        def _steady(h):
            c0 = 2 * h
            prep(c0 + 1, bufs_b)
            copy(bufs_b, sem_b).start()
            drain(c0, bufs_a, sem_a)

            @pl.when(h < NCH // 2 - 1)
            def _refill():
                prep(c0 + 2, bufs_a)
                copy(bufs_a, sem_a).start()

            drain(c0 + 1, bufs_b, sem_b)

    return interp


_interp = _make_interp()


def kernel(values, grid_latitude, grid_longitude, query_latitude, query_longitude):
    del grid_latitude, grid_longitude
    vr = jnp.roll(values, -1, axis=1)
    vd = jnp.roll(values, -1, axis=0)
    vdr = jnp.roll(vr, -1, axis=0)
    octos = jnp.stack([values, vr, vd, vdr], axis=-1).reshape(LAT * LON // 2, 8)
    return _interp(octos, query_latitude, query_longitude)


# speedup vs baseline: 13.4992x; 13.4992x over previous
import functools

import jax
import jax.numpy as jnp
from jax import lax
from jax.experimental import pallas as pl
from jax.experimental.pallas import tpu as pltpu
from jax.experimental.pallas import tpu_sc as plsc

LAT, LON, NQ = 1801, 3600, 1048576
NC, NS, L = 2, 16, 16
NW = NC * NS
QPW = NQ // NW
CH = 1024
NCH = QPW // CH
RSTEP = LON // 8
ROWS = LAT * RSTEP

_BR = 8


def _pair_body(x_ref, o_ref):
    x = x_ref[...]
    xr = jnp.concatenate([x[:, 1:], x[:, :1]], axis=1)
    a = x.reshape(_BR, RSTEP, 8)
    b = xr.reshape(_BR, RSTEP, 8)
    o_ref[...] = jnp.concatenate([a, b], axis=2).reshape(_BR, 2 * LON)


def _build_pairs(values):
    grid = (LAT + _BR - 1) // _BR
    return pl.pallas_call(
        _pair_body,
        grid=(grid,),
        in_specs=[pl.BlockSpec((_BR, LON), lambda g: (g, 0))],
        out_specs=pl.BlockSpec((_BR, 2 * LON), lambda g: (g, 0)),
        out_shape=jax.ShapeDtypeStruct((LAT, 2 * LON), jnp.float32),
    )(values)


def _buf_set():
    return (
        [pltpu.VMEM((CH,), jnp.float32)] * 2
        + [pltpu.VMEM((CH,), jnp.int32)] * 3
        + [pltpu.VMEM((CH,), jnp.float32)] * 2
        + [pltpu.VMEM((CH, 16), jnp.float32)] * 2
    )


def _make_interp():
    mesh = plsc.VectorSubcoreMesh(core_axis_name="c", subcore_axis_name="s")

    @functools.partial(
        pl.kernel,
        out_type=jax.ShapeDtypeStruct((NQ,), jnp.float32),
        mesh=mesh,
        compiler_params=pltpu.CompilerParams(
            needs_layout_passes=False, use_tc_tiling_on_sc=False),
        scratch_types=[
            _buf_set(),
            _buf_set(),
            pltpu.VMEM((CH,), jnp.float32),
            pltpu.SemaphoreType.DMA,
            pltpu.SemaphoreType.DMA,
        ],
    )
    def interp(pairs, qlat, qlon, out, bufs_a, bufs_b, res_v, sem_a, sem_b):
        wid = lax.axis_index("s") * NC + lax.axis_index("c")
        base = wid * QPW

        def prep(c, bufs):
            xq_v, yq_v, r0_v, r1_v, cb_v, t_v, u_v = bufs[:7]
            off = base + c * CH
            pltpu.sync_copy(qlat.at[pl.ds(off, CH)], xq_v)
            pltpu.sync_copy(qlon.at[pl.ds(off, CH)], yq_v)

            @pl.loop(0, CH // L, unroll=4)
            def _idx(kv):
                s = pl.ds(kv * L, L)
                fx = (xq_v[s] + 90.0) * 10.0
                fy = (yq_v[s] + 180.0) * 10.0
                ix = jnp.minimum(fx.astype(jnp.int32), LAT - 2)
                jy = jnp.minimum(fy.astype(jnp.int32), LON - 1)
                t_v[s] = fx - ix.astype(jnp.float32)
                u_v[s] = fy - jy.astype(jnp.float32)
                r0 = ix * RSTEP + (jy >> 3)
                r0_v[s] = r0
                r1_v[s] = r0 + RSTEP
                cb_v[s] = jy & 7

        def copies(bufs, sem):
            r0_v, r1_v = bufs[2], bufs[3]
            g0_v, g1_v = bufs[7], bufs[8]
            return (
                pltpu.make_async_copy(pairs.at[r0_v], g0_v, sem),
                pltpu.make_async_copy(pairs.at[r1_v], g1_v, sem),
            )

        def fire(bufs, sem):
            for cp in copies(bufs, sem):
                cp.start()

        def drain(c, bufs, sem):
            for cp in copies(bufs, sem):
                cp.wait()
            cb_v, t_v, u_v, g0_v, g1_v = bufs[4], bufs[5], bufs[6], bufs[7], bufs[8]

            @pl.loop(0, CH // L, unroll=4)
            def _blend(kv):
                s = pl.ds(kv * L, L)
                q = lax.iota(jnp.int32, L) + kv * L
                cb = cb_v[s]
                v00 = plsc.load_gather(g0_v, [q, cb])
                v01 = plsc.load_gather(g0_v, [q, cb + 8])
                v10 = plsc.load_gather(g1_v, [q, cb])
                v11 = plsc.load_gather(g1_v, [q, cb + 8])
                t = t_v[s]
                u = u_v[s]
                res_v[s] = ((1.0 - t) * (1.0 - u) * v00
                            + (1.0 - t) * u * v01
                            + t * (1.0 - u) * v10
                            + t * u * v11)

            pltpu.sync_copy(res_v, out.at[pl.ds(base + c * CH, CH)])

        prep(0, bufs_a)
        fire(bufs_a, sem_a)

        @pl.loop(0, NCH // 2)
        def _steady(h):
            c0 = 2 * h
            prep(c0 + 1, bufs_b)
            fire(bufs_b, sem_b)
            drain(c0, bufs_a, sem_a)

            @pl.when(h < NCH // 2 - 1)
            def _refill():
                prep(c0 + 2, bufs_a)
                fire(bufs_a, sem_a)

            drain(c0 + 1, bufs_b, sem_b)

    return interp


_interp = _make_interp()


def kernel(values, grid_latitude, grid_longitude, query_latitude, query_longitude):
    del grid_latitude, grid_longitude
    pairs = _build_pairs(values).reshape(ROWS, 16)
    return _interp(pairs, query_latitude, query_longitude)


# speedup vs baseline: 15.6635x; 1.1603x over previous
import functools

import jax
import jax.numpy as jnp
from jax import lax
from jax.experimental import pallas as pl
from jax.experimental.pallas import tpu as pltpu
from jax.experimental.pallas import tpu_sc as plsc

LAT, LON, NQ = 1801, 3600, 1048576
NC, NS, L = 2, 16, 16
NW = NC * NS
QPW = NQ // NW
CH = 2048
NCH = QPW // CH

NV = LAT * LON
SB = NV // 8
T2ROWS = 2 * SB
RSTEP = LON // 8
NE = 1808


def _buf_set():
    return (
        [pltpu.VMEM((CH,), jnp.float32)] * 2
        + [pltpu.VMEM((CH,), jnp.int32)] * 5
        + [pltpu.VMEM((CH,), jnp.float32)] * 2
        + [pltpu.VMEM((CH, 8), jnp.float32)] * 2
    )


def _make_interp():
    mesh = plsc.VectorSubcoreMesh(core_axis_name="c", subcore_axis_name="s")

    @functools.partial(
        pl.kernel,
        out_type=jax.ShapeDtypeStruct((NQ,), jnp.float32),
        mesh=mesh,
        compiler_params=pltpu.CompilerParams(
            needs_layout_passes=False, use_tc_tiling_on_sc=False),
        scratch_types=[
            _buf_set(),
            _buf_set(),
            pltpu.VMEM((CH,), jnp.float32),
            pltpu.VMEM((NE,), jnp.int32),
            pltpu.VMEM((NE, 8), jnp.float32),
            pltpu.SemaphoreType.DMA,
            pltpu.SemaphoreType.DMA,
        ],
    )
    def interp(t2, qlat, qlon, out, bufs_a, bufs_b, res_v, eidx_v, e8_v,
               sem_a, sem_b):
        wid = lax.axis_index("s") * NC + lax.axis_index("c")
        base = wid * QPW

        @pl.loop(0, NE // L)
        def _eidx(kv):
            eidx_v[pl.ds(kv * L, L)] = jnp.minimum(
                (lax.iota(jnp.int32, L) + kv * L) * RSTEP, (LAT - 1) * RSTEP)

        pltpu.async_copy(t2.at[eidx_v], e8_v, sem_a).wait()

        def prep(c, bufs):
            xq_v, yq_v, r0_v, r1_v, cb_v, ix_v, wr_v, t_v, u_v = bufs[:9]
            off = base + c * CH
            pltpu.sync_copy(qlat.at[pl.ds(off, CH)], xq_v)
            pltpu.sync_copy(qlon.at[pl.ds(off, CH)], yq_v)

            @pl.loop(0, CH // L, unroll=4)
            def _idx(kv):
                s = pl.ds(kv * L, L)
                fx = (xq_v[s] + 90.0) * 10.0
                fy = (yq_v[s] + 180.0) * 10.0
                ix = jnp.minimum(fx.astype(jnp.int32), LAT - 2)
                jy = jnp.minimum(fy.astype(jnp.int32), LON - 1)
                t_v[s] = fx - ix.astype(jnp.float32)
                u_v[s] = fy - jy.astype(jnp.float32)
                o = jy & 7
                shifted = o == 7
                r0 = ((ix * LON + jy) >> 3) + jnp.where(shifted, SB, 0)
                r0_v[s] = r0
                r1_v[s] = r0 + RSTEP
                cb_v[s] = jnp.where(shifted, 3, o)
                ix_v[s] = ix
                wr_v[s] = jnp.where(jy == LON - 1, 1, 0)

        def copies(bufs, sem):
            r0_v, r1_v = bufs[2], bufs[3]
            g0_v, g1_v = bufs[9], bufs[10]
            return (
                pltpu.make_async_copy(t2.at[r0_v], g0_v, sem),
                pltpu.make_async_copy(t2.at[r1_v], g1_v, sem),
            )

        def fire(bufs, sem):
            for cp in copies(bufs, sem):
                cp.start()

        def drain(c, bufs, sem):
            for cp in copies(bufs, sem):
                cp.wait()
            cb_v, ix_v, wr_v, t_v, u_v = bufs[4:9]
            g0_v, g1_v = bufs[9], bufs[10]

            @pl.loop(0, CH // L, unroll=4)
            def _blend(kv):
                s = pl.ds(kv * L, L)
                q = lax.iota(jnp.int32, L) + kv * L
                zero = jnp.zeros((L,), jnp.int32)
                cb = cb_v[s]
                wrap = wr_v[s] > 0
                ix = ix_v[s]
                v00 = plsc.load_gather(g0_v, [q, cb])
                v10 = plsc.load_gather(g1_v, [q, cb])
                v01 = jnp.where(wrap,
                                plsc.load_gather(e8_v, [ix, zero]),
                                plsc.load_gather(g0_v, [q, cb + 1]))
                v11 = jnp.where(wrap,
                                plsc.load_gather(e8_v, [ix + 1, zero]),
                                plsc.load_gather(g1_v, [q, cb + 1]))
                t = t_v[s]
                u = u_v[s]
                res_v[s] = ((1.0 - t) * (1.0 - u) * v00
                            + (1.0 - t) * u * v01
                            + t * (1.0 - u) * v10
                            + t * u * v11)

            pltpu.sync_copy(res_v, out.at[pl.ds(base + c * CH, CH)])

        prep(0, bufs_a)
        fire(bufs_a, sem_a)

        @pl.loop(0, NCH // 2)
        def _steady(h):
            c0 = 2 * h
            prep(c0 + 1, bufs_b)
            fire(bufs_b, sem_b)
            drain(c0, bufs_a, sem_a)

            @pl.when(h < NCH // 2 - 1)
            def _refill():
                prep(c0 + 2, bufs_a)
                fire(bufs_a, sem_a)

            drain(c0 + 1, bufs_b, sem_b)

    return interp


_interp = _make_interp()


def kernel(values, grid_latitude, grid_longitude, query_latitude, query_longitude):
    del grid_latitude, grid_longitude
    flat = values.reshape(NV)
    t2 = jnp.concatenate([flat, flat[4:], jnp.zeros(4, jnp.float32)])
    return _interp(t2.reshape(T2ROWS, 8), query_latitude, query_longitude)


# speedup vs baseline: 40.6130x; 2.5928x over previous
import functools

import jax
import jax.numpy as jnp
from jax import lax
from jax.experimental import pallas as pl
from jax.experimental.pallas import tpu as pltpu
from jax.experimental.pallas import tpu_sc as plsc

LAT, LON, NQ = 1801, 3600, 1048576
NC, NS, L = 2, 16, 16
NW = NC * NS
QPW = NQ // NW
CH = 2048
NCH = QPW // CH

NV = LAT * LON
RSTEP = LON // 8
NE = 1808

_BR = 256
_NA = 8
_T2LAT = 2 * _NA * _BR
SB = _NA * _BR * LON // 8
T2ROWS = _T2LAT * LON // 8


def _t2_body(xa_ref, xb_ref, o_ref):
    g = pl.program_id(0)
    xa = xa_ref[...]
    xb = xb_ref[...]
    nxt = jnp.concatenate([xa[1:, :], xb[:1, :]], axis=0)
    shifted = jnp.concatenate([xa[:, 4:], nxt[:, :4]], axis=1)
    o_ref[...] = jnp.where(g < _NA, xa, shifted)


def _build_t2(values):
    return pl.pallas_call(
        _t2_body,
        grid=(2 * _NA,),
        in_specs=[
            pl.BlockSpec((_BR, LON),
                         lambda g: (jnp.where(g < _NA, g, g - _NA), 0)),
            pl.BlockSpec((_BR, LON),
                         lambda g: (jnp.where(g < _NA, 0,
                                              jnp.minimum(g - _NA + 1, _NA - 1)), 0)),
        ],
        out_specs=pl.BlockSpec((_BR, LON), lambda g: (g, 0)),
        out_shape=jax.ShapeDtypeStruct((_T2LAT, LON), jnp.float32),
    )(values, values)


def _buf_set():
    return (
        [pltpu.VMEM((CH,), jnp.float32)] * 2
        + [pltpu.VMEM((CH,), jnp.int32)] * 5
        + [pltpu.VMEM((CH,), jnp.float32)] * 2
        + [pltpu.VMEM((CH, 8), jnp.float32)] * 2
    )


def _make_interp():
    mesh = plsc.VectorSubcoreMesh(core_axis_name="c", subcore_axis_name="s")

    @functools.partial(
        pl.kernel,
        out_type=jax.ShapeDtypeStruct((NQ,), jnp.float32),
        mesh=mesh,
        compiler_params=pltpu.CompilerParams(
            needs_layout_passes=False, use_tc_tiling_on_sc=False),
        scratch_types=[
            _buf_set(),
            _buf_set(),
            pltpu.VMEM((CH,), jnp.float32),
            pltpu.VMEM((NE,), jnp.int32),
            pltpu.VMEM((NE, 8), jnp.float32),
            pltpu.SemaphoreType.DMA,
            pltpu.SemaphoreType.DMA,
        ],
    )
    def interp(t2, qlat, qlon, out, bufs_a, bufs_b, res_v, eidx_v, e8_v,
               sem_a, sem_b):
        wid = lax.axis_index("s") * NC + lax.axis_index("c")
        base = wid * QPW

        @pl.loop(0, NE // L)
        def _eidx(kv):
            eidx_v[pl.ds(kv * L, L)] = jnp.minimum(
                (lax.iota(jnp.int32, L) + kv * L) * RSTEP, (LAT - 1) * RSTEP)

        pltpu.async_copy(t2.at[eidx_v], e8_v, sem_a).wait()

        def prep(c, bufs):
            xq_v, yq_v, r0_v, r1_v, cb_v, ix_v, wr_v, t_v, u_v = bufs[:9]
            off = base + c * CH
            pltpu.sync_copy(qlat.at[pl.ds(off, CH)], xq_v)
            pltpu.sync_copy(qlon.at[pl.ds(off, CH)], yq_v)

            @pl.loop(0, CH // L, unroll=4)
            def _idx(kv):
                s = pl.ds(kv * L, L)
                fx = (xq_v[s] + 90.0) * 10.0
                fy = (yq_v[s] + 180.0) * 10.0
                ix = jnp.minimum(fx.astype(jnp.int32), LAT - 2)
                jy = jnp.minimum(fy.astype(jnp.int32), LON - 1)
                t_v[s] = fx - ix.astype(jnp.float32)
                u_v[s] = fy - jy.astype(jnp.float32)
                o = jy & 7
                shifted = o == 7
                r0 = ((ix * LON + jy) >> 3) + jnp.where(shifted, SB, 0)
                r0_v[s] = r0
                r1_v[s] = r0 + RSTEP
                cb_v[s] = jnp.where(shifted, 3, o)
                ix_v[s] = ix
                wr_v[s] = jnp.where(jy == LON - 1, 1, 0)

        def copies(bufs, sem):
            r0_v, r1_v = bufs[2], bufs[3]
            g0_v, g1_v = bufs[9], bufs[10]
            return (
                pltpu.make_async_copy(t2.at[r0_v], g0_v, sem),
                pltpu.make_async_copy(t2.at[r1_v], g1_v, sem),
            )

        def fire(bufs, sem):
            for cp in copies(bufs, sem):
                cp.start()

        def drain(c, bufs, sem):
            for cp in copies(bufs, sem):
                cp.wait()
            cb_v, ix_v, wr_v, t_v, u_v = bufs[4:9]
            g0_v, g1_v = bufs[9], bufs[10]

            @pl.loop(0, CH // L, unroll=4)
            def _blend(kv):
                s = pl.ds(kv * L, L)
                q = lax.iota(jnp.int32, L) + kv * L
                zero = jnp.zeros((L,), jnp.int32)
                cb = cb_v[s]
                wrap = wr_v[s] > 0
                ix = ix_v[s]
                v00 = plsc.load_gather(g0_v, [q, cb])
                v10 = plsc.load_gather(g1_v, [q, cb])
                v01 = jnp.where(wrap,
                                plsc.load_gather(e8_v, [ix, zero]),
                                plsc.load_gather(g0_v, [q, cb + 1]))
                v11 = jnp.where(wrap,
                                plsc.load_gather(e8_v, [ix + 1, zero]),
                                plsc.load_gather(g1_v, [q, cb + 1]))
                t = t_v[s]
                u = u_v[s]
                res_v[s] = ((1.0 - t) * (1.0 - u) * v00
                            + (1.0 - t) * u * v01
                            + t * (1.0 - u) * v10
                            + t * u * v11)

            pltpu.sync_copy(res_v, out.at[pl.ds(base + c * CH, CH)])

        prep(0, bufs_a)
        fire(bufs_a, sem_a)

        @pl.loop(0, NCH // 2)
        def _steady(h):
            c0 = 2 * h
            prep(c0 + 1, bufs_b)
            fire(bufs_b, sem_b)
            drain(c0, bufs_a, sem_a)

            @pl.when(h < NCH // 2 - 1)
            def _refill():
                prep(c0 + 2, bufs_a)
                fire(bufs_a, sem_a)

            drain(c0 + 1, bufs_b, sem_b)

    return interp


_interp = _make_interp()


def kernel(values, grid_latitude, grid_longitude, query_latitude, query_longitude):
    del grid_latitude, grid_longitude
    t2 = _build_t2(values).reshape(T2ROWS, 8)
    return _interp(t2, query_latitude, query_longitude)
